# Initial kernel scaffold; baseline (speedup 1.0000x reference)
import jax
import jax.numpy as jnp
from jax.experimental import pallas as pl


def kernel(xyz):
    raise NotImplementedError("write your pallas kernel here")


# speedup vs baseline: 11.0301x; 11.0301x over previous
import functools

import jax
import jax.numpy as jnp
from jax import lax
from jax.experimental import pallas as pl
from jax.experimental.pallas import tpu as pltpu
from jax.experimental.pallas import tpu_sc as plsc

NUM_G = 512
GSZ = 32
PAD_D = 16
GT = 64


def _fps_body(x_ref, y_ref, z_ref, cx_ref, cy_ref, cz_ref):
    B, N = x_ref.shape
    X = x_ref[...]
    Y = y_ref[...]
    Z = z_ref[...]
    iota_n = lax.broadcasted_iota(jnp.int32, (B, N), 1)
    iota_g = lax.broadcasted_iota(jnp.int32, (B, NUM_G), 1)

    def body(i, carry):
        dists, far, ax, ay, az = carry
        m = iota_n == far
        cx = jnp.sum(jnp.where(m, X, 0.0), axis=1, keepdims=True)
        cy = jnp.sum(jnp.where(m, Y, 0.0), axis=1, keepdims=True)
        cz = jnp.sum(jnp.where(m, Z, 0.0), axis=1, keepdims=True)
        sel = iota_g == i
        ax = jnp.where(sel, cx, ax)
        ay = jnp.where(sel, cy, ay)
        az = jnp.where(sel, cz, az)
        dx = X - cx
        dy = Y - cy
        dz = Z - cz
        d = (dx * dx + dy * dy) + dz * dz
        dists = jnp.minimum(dists, d)
        mx = jnp.max(dists, axis=1, keepdims=True)
        far_new = jnp.min(jnp.where(dists == mx, iota_n, N), axis=1,
                          keepdims=True)
        return dists, far_new, ax, ay, az

    init = (jnp.full((B, N), 1e10, jnp.float32),
            jnp.zeros((B, 1), jnp.int32),
            jnp.zeros((B, NUM_G), jnp.float32),
            jnp.zeros((B, NUM_G), jnp.float32),
            jnp.zeros((B, NUM_G), jnp.float32))
    _, _, ax, ay, az = lax.fori_loop(0, NUM_G, body, init)
    cx_ref[...] = ax
    cy_ref[...] = ay
    cz_ref[...] = az


def _fps(x, y, z):
    B, _ = x.shape
    out = [jax.ShapeDtypeStruct((B, NUM_G), jnp.float32)] * 3
    return pl.pallas_call(_fps_body, out_shape=out)(x, y, z)


def _knn_body(cx_ref, cy_ref, cz_ref, x_ref, y_ref, z_ref, idx_ref):
    N = x_ref.shape[-1]
    cx = cx_ref[0]
    cy = cy_ref[0]
    cz = cz_ref[0]
    X = x_ref[0]
    Y = y_ref[0]
    Z = z_ref[0]
    dx = cx - X
    dy = cy - Y
    dz = cz - Z
    d2 = (dx * dx + dy * dy) + dz * dz
    iota = lax.broadcasted_iota(jnp.int32, (GT, N), 1)
    for k in range(GSZ):
        mn = jnp.min(d2, axis=1, keepdims=True)
        c = d2 == mn
        j = jnp.min(jnp.where(c, iota, N), axis=1, keepdims=True)
        idx_ref[0, :, pl.ds(k, 1)] = j
        d2 = jnp.where(iota == j, jnp.inf, d2)


def _knn(cx3, cy3, cz3, x3, y3, z3):
    B = x3.shape[0]
    N = x3.shape[-1]
    cspec = pl.BlockSpec((1, GT, 1), lambda b, g: (b, g, 0))
    pspec = pl.BlockSpec((1, 1, N), lambda b, g: (b, 0, 0))
    return pl.pallas_call(
        _knn_body,
        grid=(B, NUM_G // GT),
        in_specs=[cspec, cspec, cspec, pspec, pspec, pspec],
        out_specs=pl.BlockSpec((1, GT, GSZ), lambda b, g: (b, g, 0)),
        out_shape=jax.ShapeDtypeStruct((B, NUM_G, GSZ), jnp.int32),
    )(cx3, cy3, cz3, x3, y3, z3)


_NC = 2
_NW = 32


def _gather_call(xf, yf, zf, idx2, cgx, cgy, cgz, n, bsz, rpw, gpw):
    mesh = plsc.VectorSubcoreMesh(core_axis_name="c", subcore_axis_name="s")
    wpb = _NW // bsz

    @functools.partial(
        pl.kernel, mesh=mesh,
        compiler_params=pltpu.CompilerParams(needs_layout_passes=False),
        out_type=[jax.ShapeDtypeStruct((_NW, rpw), jnp.float32)] * 3,
        scratch_types=[
            pltpu.VMEM((n,), jnp.float32),
            pltpu.VMEM((n,), jnp.float32),
            pltpu.VMEM((n,), jnp.float32),
            pltpu.VMEM((rpw,), jnp.int32),
            pltpu.VMEM((gpw,), jnp.float32),
            pltpu.VMEM((gpw,), jnp.float32),
            pltpu.VMEM((gpw,), jnp.float32),
            pltpu.VMEM((rpw,), jnp.float32),
            pltpu.VMEM((rpw,), jnp.float32),
            pltpu.VMEM((rpw,), jnp.float32),
        ],
    )
    def k(xf_h, yf_h, zf_h, idx_h, cgx_h, cgy_h, cgz_h,
          ox_h, oy_h, oz_h,
          tx, ty, tz, idx_v, cvx, cvy, cvz, ox, oy, oz):
        wid = lax.axis_index("s") * _NC + lax.axis_index("c")
        b = wid // wpb
        pltpu.sync_copy(xf_h.at[pl.ds(b * n, n)], tx)
        pltpu.sync_copy(yf_h.at[pl.ds(b * n, n)], ty)
        pltpu.sync_copy(zf_h.at[pl.ds(b * n, n)], tz)
        pltpu.sync_copy(idx_h.at[wid], idx_v)
        pltpu.sync_copy(cgx_h.at[pl.ds(wid * gpw, gpw)], cvx)
        pltpu.sync_copy(cgy_h.at[pl.ds(wid * gpw, gpw)], cvy)
        pltpu.sync_copy(cgz_h.at[pl.ds(wid * gpw, gpw)], cvz)

        def gbody(g, _):
            gs = jnp.full((16,), g, jnp.int32)
            cx = plsc.load_gather(cvx, [gs])
            cy = plsc.load_gather(cvy, [gs])
            cz = plsc.load_gather(cvz, [gs])
            for h in range(GSZ // 16):
                base = g * GSZ + h * 16
                iv = idx_v[pl.ds(base, 16)]
                ox[pl.ds(base, 16)] = plsc.load_gather(tx, [iv]) - cx
                oy[pl.ds(base, 16)] = plsc.load_gather(ty, [iv]) - cy
                oz[pl.ds(base, 16)] = plsc.load_gather(tz, [iv]) - cz
            return 0

        lax.fori_loop(0, gpw, gbody, 0)
        pltpu.sync_copy(ox, ox_h.at[wid])
        pltpu.sync_copy(oy, oy_h.at[wid])
        pltpu.sync_copy(oz, oz_h.at[wid])

    return k(xf, yf, zf, idx2, cgx, cgy, cgz)


def kernel(xyz):
    B, N, _ = xyz.shape
    x = xyz[:, :, 0]
    y = xyz[:, :, 1]
    z = xyz[:, :, 2]
    cx, cy, cz = _fps(x, y, z)
    center = jnp.stack([cx, cy, cz], axis=-1)

    idx = _knn(cx[:, :, None], cy[:, :, None], cz[:, :, None],
               x[:, None, :], y[:, None, :], z[:, None, :])

    rows = B * NUM_G * GSZ
    rpw = rows // _NW
    gpw = (B * NUM_G) // _NW
    ox, oy, oz = _gather_call(
        x.reshape(-1), y.reshape(-1), z.reshape(-1),
        idx.reshape(_NW, rpw),
        cx.reshape(-1), cy.reshape(-1), cz.reshape(-1),
        N, B, rpw, gpw)
    neighborhood = jnp.stack(
        [ox.reshape(-1), oy.reshape(-1), oz.reshape(-1)],
        axis=-1).reshape(B, NUM_G, GSZ, 3)
    return neighborhood, center


# speedup vs baseline: 11.1582x; 1.0116x over previous
import functools

import jax
import jax.numpy as jnp
from jax import lax
from jax.experimental import pallas as pl
from jax.experimental.pallas import tpu as pltpu
from jax.experimental.pallas import tpu_sc as plsc

NUM_G = 512
GSZ = 32
PAD_D = 16
GT = 64


def _fps_body(x_ref, y_ref, z_ref, cx_ref, cy_ref, cz_ref):
    B, N = x_ref.shape
    X = x_ref[...]
    Y = y_ref[...]
    Z = z_ref[...]
    iota_n = lax.broadcasted_iota(jnp.int32, (B, N), 1)
    iota_g = lax.broadcasted_iota(jnp.int32, (B, NUM_G), 1)

    def body(i, carry):
        dists, far, ax, ay, az = carry
        m = iota_n == far
        cx = jnp.sum(jnp.where(m, X, 0.0), axis=1, keepdims=True)
        cy = jnp.sum(jnp.where(m, Y, 0.0), axis=1, keepdims=True)
        cz = jnp.sum(jnp.where(m, Z, 0.0), axis=1, keepdims=True)
        sel = iota_g == i
        ax = jnp.where(sel, cx, ax)
        ay = jnp.where(sel, cy, ay)
        az = jnp.where(sel, cz, az)
        dx = X - cx
        dy = Y - cy
        dz = Z - cz
        d = (dx * dx + dy * dy) + dz * dz
        dists = jnp.minimum(dists, d)
        mx = jnp.max(dists, axis=1, keepdims=True)
        far_new = jnp.min(jnp.where(dists == mx, iota_n, N), axis=1,
                          keepdims=True)
        return dists, far_new, ax, ay, az

    init = (jnp.full((B, N), 1e10, jnp.float32),
            jnp.zeros((B, 1), jnp.int32),
            jnp.zeros((B, NUM_G), jnp.float32),
            jnp.zeros((B, NUM_G), jnp.float32),
            jnp.zeros((B, NUM_G), jnp.float32))
    _, _, ax, ay, az = lax.fori_loop(0, NUM_G, body, init)
    cx_ref[...] = ax
    cy_ref[...] = ay
    cz_ref[...] = az


def _fps(x, y, z):
    B, _ = x.shape
    out = [jax.ShapeDtypeStruct((B, NUM_G), jnp.float32)] * 3
    return pl.pallas_call(_fps_body, out_shape=out)(x, y, z)


def _knn_body(cx_ref, cy_ref, cz_ref, x_ref, y_ref, z_ref, idx_ref):
    N = x_ref.shape[-1]
    NCH = N // 128
    cx = cx_ref[0]
    cy = cy_ref[0]
    cz = cz_ref[0]
    X = x_ref[0]
    Y = y_ref[0]
    Z = z_ref[0]
    dx = cx - X
    dy = cy - Y
    dz = cz - Z
    d2 = (dx * dx + dy * dy) + dz * dz
    d3 = d2.reshape(GT, NCH, 128)

    S = jnp.min(d3, axis=2)
    iota_c = lax.broadcasted_iota(jnp.int32, (GT, NCH), 1)
    cids = []
    for _ in range(GSZ):
        m = jnp.min(S, axis=1, keepdims=True)
        cid = jnp.min(jnp.where(S == m, iota_c, NCH), axis=1, keepdims=True)
        cids.append(cid)
        S = jnp.where(iota_c == cid, jnp.inf, S)
    segid = jnp.concatenate(cids, axis=1)

    iota_c3 = lax.broadcasted_iota(jnp.int32, (GT, GSZ, NCH), 2)
    P = (segid[:, :, None] == iota_c3).astype(jnp.float32)
    cand = lax.dot_general(P, d3, (((2,), (1,)), ((0,), (0,))),
                           preferred_element_type=jnp.float32)
    cand = cand.reshape(GT, GSZ * 128)
    oidx = (segid[:, :, None] * 128
            + lax.broadcasted_iota(jnp.int32, (GT, GSZ, 128), 2)
            ).reshape(GT, GSZ * 128)

    for k in range(GSZ):
        mn = jnp.min(cand, axis=1, keepdims=True)
        c = cand == mn
        j = jnp.min(jnp.where(c, oidx, N), axis=1, keepdims=True)
        idx_ref[0, :, pl.ds(k, 1)] = j
        cand = jnp.where(oidx == j, jnp.inf, cand)


def _knn(cx3, cy3, cz3, x3, y3, z3):
    B = x3.shape[0]
    N = x3.shape[-1]
    cspec = pl.BlockSpec((1, GT, 1), lambda b, g: (b, g, 0))
    pspec = pl.BlockSpec((1, 1, N), lambda b, g: (b, 0, 0))
    return pl.pallas_call(
        _knn_body,
        grid=(B, NUM_G // GT),
        in_specs=[cspec, cspec, cspec, pspec, pspec, pspec],
        out_specs=pl.BlockSpec((1, GT, GSZ), lambda b, g: (b, g, 0)),
        out_shape=jax.ShapeDtypeStruct((B, NUM_G, GSZ), jnp.int32),
    )(cx3, cy3, cz3, x3, y3, z3)


_NC = 2
_NW = 32


def _gather_call(xf, yf, zf, idx2, cgx, cgy, cgz, n, bsz, rpw, gpw):
    mesh = plsc.VectorSubcoreMesh(core_axis_name="c", subcore_axis_name="s")
    wpb = _NW // bsz

    @functools.partial(
        pl.kernel, mesh=mesh,
        compiler_params=pltpu.CompilerParams(needs_layout_passes=False),
        out_type=[jax.ShapeDtypeStruct((_NW, rpw), jnp.float32)] * 3,
        scratch_types=[
            pltpu.VMEM((n,), jnp.float32),
            pltpu.VMEM((n,), jnp.float32),
            pltpu.VMEM((n,), jnp.float32),
            pltpu.VMEM((rpw,), jnp.int32),
            pltpu.VMEM((gpw,), jnp.float32),
            pltpu.VMEM((gpw,), jnp.float32),
            pltpu.VMEM((gpw,), jnp.float32),
            pltpu.VMEM((rpw,), jnp.float32),
            pltpu.VMEM((rpw,), jnp.float32),
            pltpu.VMEM((rpw,), jnp.float32),
        ],
    )
    def k(xf_h, yf_h, zf_h, idx_h, cgx_h, cgy_h, cgz_h,
          ox_h, oy_h, oz_h,
          tx, ty, tz, idx_v, cvx, cvy, cvz, ox, oy, oz):
        wid = lax.axis_index("s") * _NC + lax.axis_index("c")
        b = wid // wpb
        pltpu.sync_copy(xf_h.at[pl.ds(b * n, n)], tx)
        pltpu.sync_copy(yf_h.at[pl.ds(b * n, n)], ty)
        pltpu.sync_copy(zf_h.at[pl.ds(b * n, n)], tz)
        pltpu.sync_copy(idx_h.at[wid], idx_v)
        pltpu.sync_copy(cgx_h.at[pl.ds(wid * gpw, gpw)], cvx)
        pltpu.sync_copy(cgy_h.at[pl.ds(wid * gpw, gpw)], cvy)
        pltpu.sync_copy(cgz_h.at[pl.ds(wid * gpw, gpw)], cvz)

        def gbody(g, _):
            gs = jnp.full((16,), g, jnp.int32)
            cx = plsc.load_gather(cvx, [gs])
            cy = plsc.load_gather(cvy, [gs])
            cz = plsc.load_gather(cvz, [gs])
            for h in range(GSZ // 16):
                base = g * GSZ + h * 16
                iv = idx_v[pl.ds(base, 16)]
                ox[pl.ds(base, 16)] = plsc.load_gather(tx, [iv]) - cx
                oy[pl.ds(base, 16)] = plsc.load_gather(ty, [iv]) - cy
                oz[pl.ds(base, 16)] = plsc.load_gather(tz, [iv]) - cz
            return 0

        lax.fori_loop(0, gpw, gbody, 0)
        pltpu.sync_copy(ox, ox_h.at[wid])
        pltpu.sync_copy(oy, oy_h.at[wid])
        pltpu.sync_copy(oz, oz_h.at[wid])

    return k(xf, yf, zf, idx2, cgx, cgy, cgz)


def kernel(xyz):
    B, N, _ = xyz.shape
    x = xyz[:, :, 0]
    y = xyz[:, :, 1]
    z = xyz[:, :, 2]
    cx, cy, cz = _fps(x, y, z)
    center = jnp.stack([cx, cy, cz], axis=-1)

    idx = _knn(cx[:, :, None], cy[:, :, None], cz[:, :, None],
               x[:, None, :], y[:, None, :], z[:, None, :])

    rows = B * NUM_G * GSZ
    rpw = rows // _NW
    gpw = (B * NUM_G) // _NW
    ox, oy, oz = _gather_call(
        x.reshape(-1), y.reshape(-1), z.reshape(-1),
        idx.reshape(_NW, rpw),
        cx.reshape(-1), cy.reshape(-1), cz.reshape(-1),
        N, B, rpw, gpw)
    neighborhood = jnp.stack(
        [ox.reshape(-1), oy.reshape(-1), oz.reshape(-1)],
        axis=-1).reshape(B, NUM_G, GSZ, 3)
    return neighborhood, center


# speedup vs baseline: 11.2958x; 1.0123x over previous
import functools

import jax
import jax.numpy as jnp
from jax import lax
from jax.experimental import pallas as pl
from jax.experimental.pallas import tpu as pltpu
from jax.experimental.pallas import tpu_sc as plsc

NUM_G = 512
GSZ = 32
PAD_D = 16
GT = 64


def _fps_body(x_ref, y_ref, z_ref, cx_ref, cy_ref, cz_ref):
    B, N = x_ref.shape
    X = x_ref[...]
    Y = y_ref[...]
    Z = z_ref[...]
    iota_n = lax.broadcasted_iota(jnp.int32, (B, N), 1)
    iota_g = lax.broadcasted_iota(jnp.int32, (B, NUM_G), 1)

    def argmax_gather(dists):
        v, ix, gx, gy, gz = dists, iota_n, X, Y, Z
        w = N
        while w > 128:
            h = w // 2
            c = v[:, h:] > v[:, :h]
            v = jnp.where(c, v[:, h:], v[:, :h])
            ix = jnp.where(c, ix[:, h:], ix[:, :h])
            gx = jnp.where(c, gx[:, h:], gx[:, :h])
            gy = jnp.where(c, gy[:, h:], gy[:, :h])
            gz = jnp.where(c, gz[:, h:], gz[:, :h])
            w = h
        m = jnp.max(v, axis=1, keepdims=True)
        far = jnp.min(jnp.where(v == m, ix, N), axis=1, keepdims=True)
        one = (v == m) & (ix == far)
        cx = jnp.sum(jnp.where(one, gx, 0.0), axis=1, keepdims=True)
        cy = jnp.sum(jnp.where(one, gy, 0.0), axis=1, keepdims=True)
        cz = jnp.sum(jnp.where(one, gz, 0.0), axis=1, keepdims=True)
        return cx, cy, cz

    def body(i, carry):
        dists, cx, cy, cz, ax, ay, az = carry
        sel = iota_g == i
        ax = jnp.where(sel, cx, ax)
        ay = jnp.where(sel, cy, ay)
        az = jnp.where(sel, cz, az)
        dx = X - cx
        dy = Y - cy
        dz = Z - cz
        d = (dx * dx + dy * dy) + dz * dz
        dists = jnp.minimum(dists, d)
        cx, cy, cz = argmax_gather(dists)
        return dists, cx, cy, cz, ax, ay, az

    x0 = jnp.sum(jnp.where(iota_n == 0, X, 0.0), axis=1, keepdims=True)
    y0 = jnp.sum(jnp.where(iota_n == 0, Y, 0.0), axis=1, keepdims=True)
    z0 = jnp.sum(jnp.where(iota_n == 0, Z, 0.0), axis=1, keepdims=True)
    init = (jnp.full((B, N), 1e10, jnp.float32), x0, y0, z0,
            jnp.zeros((B, NUM_G), jnp.float32),
            jnp.zeros((B, NUM_G), jnp.float32),
            jnp.zeros((B, NUM_G), jnp.float32))
    _, _, _, _, ax, ay, az = lax.fori_loop(0, NUM_G, body, init)
    cx_ref[...] = ax
    cy_ref[...] = ay
    cz_ref[...] = az


def _fps(x, y, z):
    B, _ = x.shape
    out = [jax.ShapeDtypeStruct((B, NUM_G), jnp.float32)] * 3
    return pl.pallas_call(_fps_body, out_shape=out)(x, y, z)


def _knn_body(cx_ref, cy_ref, cz_ref, x_ref, y_ref, z_ref, idx_ref):
    N = x_ref.shape[-1]
    NCH = N // 128
    cx = cx_ref[0]
    cy = cy_ref[0]
    cz = cz_ref[0]
    X = x_ref[0]
    Y = y_ref[0]
    Z = z_ref[0]
    dx = cx - X
    dy = cy - Y
    dz = cz - Z
    d2 = (dx * dx + dy * dy) + dz * dz
    iota = lax.broadcasted_iota(jnp.int32, (GT, N), 1)
    for k in range(GSZ):
        mn = jnp.min(d2, axis=1, keepdims=True)
        c = d2 == mn
        j = jnp.min(jnp.where(c, iota, N), axis=1, keepdims=True)
        idx_ref[0, :, pl.ds(k, 1)] = j
        d2 = jnp.where(iota == j, jnp.inf, d2)


def _knn(cx3, cy3, cz3, x3, y3, z3):
    B = x3.shape[0]
    N = x3.shape[-1]
    cspec = pl.BlockSpec((1, GT, 1), lambda b, g: (b, g, 0))
    pspec = pl.BlockSpec((1, 1, N), lambda b, g: (b, 0, 0))
    return pl.pallas_call(
        _knn_body,
        grid=(B, NUM_G // GT),
        in_specs=[cspec, cspec, cspec, pspec, pspec, pspec],
        out_specs=pl.BlockSpec((1, GT, GSZ), lambda b, g: (b, g, 0)),
        out_shape=jax.ShapeDtypeStruct((B, NUM_G, GSZ), jnp.int32),
    )(cx3, cy3, cz3, x3, y3, z3)


_NC = 2
_NW = 32


def _gather_call(xf, yf, zf, idx2, cgx, cgy, cgz, n, bsz, rpw, gpw):
    mesh = plsc.VectorSubcoreMesh(core_axis_name="c", subcore_axis_name="s")
    wpb = _NW // bsz

    @functools.partial(
        pl.kernel, mesh=mesh,
        compiler_params=pltpu.CompilerParams(needs_layout_passes=False),
        out_type=[jax.ShapeDtypeStruct((_NW, rpw), jnp.float32)] * 3,
        scratch_types=[
            pltpu.VMEM((n,), jnp.float32),
            pltpu.VMEM((n,), jnp.float32),
            pltpu.VMEM((n,), jnp.float32),
            pltpu.VMEM((rpw,), jnp.int32),
            pltpu.VMEM((gpw,), jnp.float32),
            pltpu.VMEM((gpw,), jnp.float32),
            pltpu.VMEM((gpw,), jnp.float32),
            pltpu.VMEM((rpw,), jnp.float32),
            pltpu.VMEM((rpw,), jnp.float32),
            pltpu.VMEM((rpw,), jnp.float32),
        ],
    )
    def k(xf_h, yf_h, zf_h, idx_h, cgx_h, cgy_h, cgz_h,
          ox_h, oy_h, oz_h,
          tx, ty, tz, idx_v, cvx, cvy, cvz, ox, oy, oz):
        wid = lax.axis_index("s") * _NC + lax.axis_index("c")
        b = wid // wpb
        pltpu.sync_copy(xf_h.at[pl.ds(b * n, n)], tx)
        pltpu.sync_copy(yf_h.at[pl.ds(b * n, n)], ty)
        pltpu.sync_copy(zf_h.at[pl.ds(b * n, n)], tz)
        pltpu.sync_copy(idx_h.at[wid], idx_v)
        pltpu.sync_copy(cgx_h.at[pl.ds(wid * gpw, gpw)], cvx)
        pltpu.sync_copy(cgy_h.at[pl.ds(wid * gpw, gpw)], cvy)
        pltpu.sync_copy(cgz_h.at[pl.ds(wid * gpw, gpw)], cvz)

        def gbody(g, _):
            gs = jnp.full((16,), g, jnp.int32)
            cx = plsc.load_gather(cvx, [gs])
            cy = plsc.load_gather(cvy, [gs])
            cz = plsc.load_gather(cvz, [gs])
            for h in range(GSZ // 16):
                base = g * GSZ + h * 16
                iv = idx_v[pl.ds(base, 16)]
                ox[pl.ds(base, 16)] = plsc.load_gather(tx, [iv]) - cx
                oy[pl.ds(base, 16)] = plsc.load_gather(ty, [iv]) - cy
                oz[pl.ds(base, 16)] = plsc.load_gather(tz, [iv]) - cz
            return 0

        lax.fori_loop(0, gpw, gbody, 0)
        pltpu.sync_copy(ox, ox_h.at[wid])
        pltpu.sync_copy(oy, oy_h.at[wid])
        pltpu.sync_copy(oz, oz_h.at[wid])

    return k(xf, yf, zf, idx2, cgx, cgy, cgz)


def kernel(xyz):
    B, N, _ = xyz.shape
    x = xyz[:, :, 0]
    y = xyz[:, :, 1]
    z = xyz[:, :, 2]
    cx, cy, cz = _fps(x, y, z)
    center = jnp.stack([cx, cy, cz], axis=-1)

    idx = _knn(cx[:, :, None], cy[:, :, None], cz[:, :, None],
               x[:, None, :], y[:, None, :], z[:, None, :])

    rows = B * NUM_G * GSZ
    rpw = rows // _NW
    gpw = (B * NUM_G) // _NW
    ox, oy, oz = _gather_call(
        x.reshape(-1), y.reshape(-1), z.reshape(-1),
        idx.reshape(_NW, rpw),
        cx.reshape(-1), cy.reshape(-1), cz.reshape(-1),
        N, B, rpw, gpw)
    neighborhood = jnp.stack(
        [ox.reshape(-1), oy.reshape(-1), oz.reshape(-1)],
        axis=-1).reshape(B, NUM_G, GSZ, 3)
    return neighborhood, center
